# baseline (speedup 1.0000x reference)
import jax
import jax.numpy as jnp
from jax import lax
from jax.experimental import pallas as pl
from jax.experimental.pallas import tpu as pltpu

_K = 200
_B, _C, _H, _W = 16, 96, 224, 224
_C_BLK = 12
_J = _C // _C_BLK
_BITS_HI = 0x7F800001


def _mse_topk_kernel(pred_ref, targ_ref, out_ref, acc_ref):
    j = pl.program_id(1)

    d = targ_ref[0] - pred_ref[0]
    part = jnp.sum(d * d, axis=0)

    @pl.when(j == 0)
    def _first():
        acc_ref[:, :] = part

    @pl.when(j > 0)
    def _rest():
        acc_ref[:, :] = acc_ref[:, :] + part

    @pl.when(j == _J - 1)
    def _finish():
        v = acc_ref[:, :]
        vi = lax.bitcast_convert_type(v, jnp.int32)

        def body(_, carry):
            lo, hi = carry
            mid = lo + (hi - lo) // 2
            ge = jnp.sum((vi >= mid).astype(jnp.int32)) >= _K
            return jnp.where(ge, mid, lo), jnp.where(ge, hi, mid)

        lo, _hi = lax.fori_loop(0, 31, body, (jnp.int32(0), jnp.int32(_BITS_HI)))
        t = lax.bitcast_convert_type(lo, jnp.float32)

        gt = vi > lo
        cnt_gt = jnp.sum(gt.astype(jnp.int32))
        sum_gt = jnp.sum(jnp.where(gt, v, 0.0))
        out_ref[0, 0, 0] = sum_gt + (_K - cnt_gt).astype(jnp.float32) * t


def kernel(pred, target):
    spec = pl.BlockSpec((1, _C_BLK, _H, _W), lambda b, j: (b, j, 0, 0))
    sums = pl.pallas_call(
        _mse_topk_kernel,
        grid=(_B, _J),
        in_specs=[spec, spec],
        out_specs=pl.BlockSpec((1, 1, 1), lambda b, j: (b, 0, 0),
                               memory_space=pltpu.SMEM),
        out_shape=jax.ShapeDtypeStruct((_B, 1, 1), jnp.float32),
        scratch_shapes=[pltpu.VMEM((_H, _W), jnp.float32)],
        compiler_params=pltpu.CompilerParams(
            dimension_semantics=("parallel", "arbitrary")),
    )(pred, target)
    return jnp.sum(sums) / (_B * _K)


# speedup vs baseline: 1.1291x; 1.1291x over previous
import jax
import jax.numpy as jnp
from jax import lax
from jax.experimental import pallas as pl
from jax.experimental.pallas import tpu as pltpu

_K = 200
_B, _C, _H, _W = 16, 96, 224, 224
_C_BLK = 48
_J = _C // _C_BLK
_BITS_HI = 0x7F800001


def _mse_topk_kernel(pred_ref, targ_ref, out_ref, acc_ref):
    j = pl.program_id(1)

    d = targ_ref[0] - pred_ref[0]
    part = jnp.sum(d * d, axis=0)

    @pl.when(j == 0)
    def _first():
        acc_ref[:, :] = part

    @pl.when(j > 0)
    def _rest():
        acc_ref[:, :] = acc_ref[:, :] + part

    @pl.when(j == _J - 1)
    def _finish():
        v = acc_ref[:, :]
        vi = lax.bitcast_convert_type(v, jnp.int32)

        def body(_, carry):
            lo, hi = carry
            mid = lo + (hi - lo) // 2
            ge = jnp.sum((vi >= mid).astype(jnp.int32)) >= _K
            return jnp.where(ge, mid, lo), jnp.where(ge, hi, mid)

        lo, _hi = lax.fori_loop(0, 31, body, (jnp.int32(0), jnp.int32(_BITS_HI)))
        t = lax.bitcast_convert_type(lo, jnp.float32)

        gt = vi > lo
        cnt_gt = jnp.sum(gt.astype(jnp.int32))
        sum_gt = jnp.sum(jnp.where(gt, v, 0.0))
        out_ref[0, 0, 0] = sum_gt + (_K - cnt_gt).astype(jnp.float32) * t


def kernel(pred, target):
    spec = pl.BlockSpec((1, _C_BLK, _H, _W), lambda b, j: (b, j, 0, 0))
    sums = pl.pallas_call(
        _mse_topk_kernel,
        grid=(_B, _J),
        in_specs=[spec, spec],
        out_specs=pl.BlockSpec((1, 1, 1), lambda b, j: (b, 0, 0),
                               memory_space=pltpu.SMEM),
        out_shape=jax.ShapeDtypeStruct((_B, 1, 1), jnp.float32),
        scratch_shapes=[pltpu.VMEM((_H, _W), jnp.float32)],
        compiler_params=pltpu.CompilerParams(
            dimension_semantics=("parallel", "arbitrary")),
    )(pred, target)
    return jnp.sum(sums) / (_B * _K)


# speedup vs baseline: 1.1564x; 1.0242x over previous
import jax
import jax.numpy as jnp
from jax import lax
from jax.experimental import pallas as pl
from jax.experimental.pallas import tpu as pltpu

_K = 200
_B, _C, _H, _W = 16, 96, 224, 224
_C_BLK = 48
_J = _C // _C_BLK
_BITS_HI = 0x7F800001


def _mse_topk_kernel(pred_a, pred_b, targ_a, targ_b, out_ref, acc_ref):
    j = pl.program_id(1)

    da = targ_a[0] - pred_a[0]
    db = targ_b[0] - pred_b[0]
    part = jnp.sum(da * da, axis=0) + jnp.sum(db * db, axis=0)

    @pl.when(j == 0)
    def _first():
        acc_ref[:, :] = part

    @pl.when(j > 0)
    def _rest():
        acc_ref[:, :] = acc_ref[:, :] + part

    @pl.when(j == _J - 1)
    def _finish():
        v = acc_ref[:, :]
        vi = lax.bitcast_convert_type(v, jnp.int32)

        def body(_, carry):
            lo, hi = carry
            mid = lo + (hi - lo) // 2
            ge = jnp.sum((vi >= mid).astype(jnp.int32)) >= _K
            return jnp.where(ge, mid, lo), jnp.where(ge, hi, mid)

        lo, _hi = lax.fori_loop(0, 31, body, (jnp.int32(0), jnp.int32(_BITS_HI)))
        t = lax.bitcast_convert_type(lo, jnp.float32)

        gt = vi > lo
        cnt_gt = jnp.sum(gt.astype(jnp.int32))
        sum_gt = jnp.sum(jnp.where(gt, v, 0.0))
        out_ref[0, 0, 0] = sum_gt + (_K - cnt_gt).astype(jnp.float32) * t


def kernel(pred, target):
    half = _C_BLK // 2
    spec_a = pl.BlockSpec((1, half, _H, _W), lambda b, j: (b, 2 * j, 0, 0))
    spec_b = pl.BlockSpec((1, half, _H, _W), lambda b, j: (b, 2 * j + 1, 0, 0))
    sums = pl.pallas_call(
        _mse_topk_kernel,
        grid=(_B, _J),
        in_specs=[spec_a, spec_b, spec_a, spec_b],
        out_specs=pl.BlockSpec((1, 1, 1), lambda b, j: (b, 0, 0),
                               memory_space=pltpu.SMEM),
        out_shape=jax.ShapeDtypeStruct((_B, 1, 1), jnp.float32),
        scratch_shapes=[pltpu.VMEM((_H, _W), jnp.float32)],
        compiler_params=pltpu.CompilerParams(
            dimension_semantics=("parallel", "arbitrary")),
    )(pred, pred, target, target)
    return jnp.sum(sums) / (_B * _K)


# speedup vs baseline: 1.1823x; 1.0225x over previous
import jax
import jax.numpy as jnp
from jax import lax
from jax.experimental import pallas as pl
from jax.experimental.pallas import tpu as pltpu

_K = 200
_B, _C, _H, _W = 16, 96, 224, 224
_C_BLK = 48
_J = _C // _C_BLK
_N_OPS = 4
_BITS_HI = 0x7F800001


def _mse_topk_kernel(*refs):
    preds = refs[:_N_OPS]
    targs = refs[_N_OPS:2 * _N_OPS]
    out_ref = refs[2 * _N_OPS]
    acc_ref = refs[2 * _N_OPS + 1]
    j = pl.program_id(1)

    part = jnp.zeros((_H, _W), jnp.float32)
    for p_ref, t_ref in zip(preds, targs):
        d = t_ref[0] - p_ref[0]
        part = part + jnp.sum(d * d, axis=0)

    @pl.when(j == 0)
    def _first():
        acc_ref[:, :] = part

    @pl.when(j > 0)
    def _rest():
        acc_ref[:, :] = acc_ref[:, :] + part

    @pl.when(j == _J - 1)
    def _finish():
        v = acc_ref[:, :]
        vi = lax.bitcast_convert_type(v, jnp.int32)

        def body(_, carry):
            lo, hi = carry
            mid = lo + (hi - lo) // 2
            ge = jnp.sum((vi >= mid).astype(jnp.int32)) >= _K
            return jnp.where(ge, mid, lo), jnp.where(ge, hi, mid)

        lo, _hi = lax.fori_loop(0, 31, body, (jnp.int32(0), jnp.int32(_BITS_HI)))
        t = lax.bitcast_convert_type(lo, jnp.float32)

        gt = vi > lo
        cnt_gt = jnp.sum(gt.astype(jnp.int32))
        sum_gt = jnp.sum(jnp.where(gt, v, 0.0))
        out_ref[0, 0, 0] = sum_gt + (_K - cnt_gt).astype(jnp.float32) * t


def kernel(pred, target):
    sub = _C_BLK // _N_OPS
    specs = [
        pl.BlockSpec((1, sub, _H, _W),
                     lambda b, j, i=i: (b, _N_OPS * j + i, 0, 0))
        for i in range(_N_OPS)
    ]
    sums = pl.pallas_call(
        _mse_topk_kernel,
        grid=(_B, _J),
        in_specs=specs + specs,
        out_specs=pl.BlockSpec((1, 1, 1), lambda b, j: (b, 0, 0),
                               memory_space=pltpu.SMEM),
        out_shape=jax.ShapeDtypeStruct((_B, 1, 1), jnp.float32),
        scratch_shapes=[pltpu.VMEM((_H, _W), jnp.float32)],
        compiler_params=pltpu.CompilerParams(
            dimension_semantics=("parallel", "arbitrary")),
    )(*([pred] * _N_OPS + [target] * _N_OPS))
    return jnp.sum(sums) / (_B * _K)


# speedup vs baseline: 1.2002x; 1.0151x over previous
import jax
import jax.numpy as jnp
from jax import lax
from jax.experimental import pallas as pl
from jax.experimental.pallas import tpu as pltpu

_K = 200
_B, _C, _H, _W = 16, 96, 224, 224
_C_BLK = 48
_J = _C // _C_BLK
_N_OPS = 8
_BITS_HI = 0x7F800001


def _mse_topk_kernel(*refs):
    preds = refs[:_N_OPS]
    targs = refs[_N_OPS:2 * _N_OPS]
    out_ref = refs[2 * _N_OPS]
    acc_ref = refs[2 * _N_OPS + 1]
    j = pl.program_id(1)

    part = jnp.zeros((_H, _W), jnp.float32)
    for p_ref, t_ref in zip(preds, targs):
        d = t_ref[0] - p_ref[0]
        part = part + jnp.sum(d * d, axis=0)

    @pl.when(j == 0)
    def _first():
        acc_ref[:, :] = part

    @pl.when(j > 0)
    def _rest():
        acc_ref[:, :] = acc_ref[:, :] + part

    @pl.when(j == _J - 1)
    def _finish():
        v = acc_ref[:, :]
        vi = lax.bitcast_convert_type(v, jnp.int32)

        def body(_, carry):
            lo, hi = carry
            mid = lo + (hi - lo) // 2
            ge = jnp.sum((vi >= mid).astype(jnp.int32)) >= _K
            return jnp.where(ge, mid, lo), jnp.where(ge, hi, mid)

        lo, _hi = lax.fori_loop(0, 31, body, (jnp.int32(0), jnp.int32(_BITS_HI)))
        t = lax.bitcast_convert_type(lo, jnp.float32)

        gt = vi > lo
        cnt_gt = jnp.sum(gt.astype(jnp.int32))
        sum_gt = jnp.sum(jnp.where(gt, v, 0.0))
        out_ref[0, 0, 0] = sum_gt + (_K - cnt_gt).astype(jnp.float32) * t


def kernel(pred, target):
    sub = _C_BLK // _N_OPS
    specs = [
        pl.BlockSpec((1, sub, _H, _W),
                     lambda b, j, i=i: (b, _N_OPS * j + i, 0, 0))
        for i in range(_N_OPS)
    ]
    sums = pl.pallas_call(
        _mse_topk_kernel,
        grid=(_B, _J),
        in_specs=specs + specs,
        out_specs=pl.BlockSpec((1, 1, 1), lambda b, j: (b, 0, 0),
                               memory_space=pltpu.SMEM),
        out_shape=jax.ShapeDtypeStruct((_B, 1, 1), jnp.float32),
        scratch_shapes=[pltpu.VMEM((_H, _W), jnp.float32)],
        compiler_params=pltpu.CompilerParams(
            dimension_semantics=("parallel", "arbitrary")),
    )(*([pred] * _N_OPS + [target] * _N_OPS))
    return jnp.sum(sums) / (_B * _K)
